# baseline (speedup 1.0000x reference)
import jax
import jax.numpy as jnp
from jax.experimental import pallas as pl
from jax.experimental.pallas import tpu as pltpu

B = 64
C = 16
K = 1024
E = 256
CW = C * E
Z = 512


def _leaky(v):
    return jnp.where(v >= 0, v, 0.2 * v)


def _mm(a, b):
    return jax.lax.dot_general(a, b, (((1,), (1,)), ((), ())),
                               preferred_element_type=jnp.float32)


def _fwd(xr_hbm, book_hbm, b2_ref,
         we1_hbm, be1_ref, we2_hbm, be2_ref, wef_hbm, bef_ref,
         wi1_hbm, bi1_ref, wp1_hbm, bp1_ref, wp2_hbm, bp2_ref,
         wq1_hbm, bq1_ref, wq2_hbm, bq2_ref,
         wd1_hbm, bd1_ref, wd2_hbm, bd2_ref,
         cw_ref, dist_ref, idx_ref, mu_ref, lv_ref, plv_ref,
         dmu_ref, dlv_ref,
         xr_v, we1_v, we2_v, wef_v, wi1_v, wp1_v, wp2_v, wq1_v, wq2_v,
         wd1_v, wd2_v, book_v, sems):
    pairs = [(xr_hbm, xr_v), (we1_hbm, we1_v), (we2_hbm, we2_v),
             (wef_hbm, wef_v), (wi1_hbm, wi1_v), (wp1_hbm, wp1_v),
             (wp2_hbm, wp2_v), (wq1_hbm, wq1_v), (wq2_hbm, wq2_v),
             (wd1_hbm, wd1_v), (wd2_hbm, wd2_v)]
    cps = [pltpu.make_async_copy(s, d, sems.at[i])
           for i, (s, d) in enumerate(pairs)]
    bookcps = [pltpu.make_async_copy(book_hbm.at[c], book_v.at[c],
                                     sems.at[len(pairs) + c])
               for c in range(C)]
    for cp in cps:
        cp.start()
    for cp in bookcps:
        cp.start()

    cps[0].wait()
    cps[1].wait()
    h1 = _leaky(_mm(xr_v[...], we1_v[...]) + be1_ref[...])
    cps[2].wait()
    h2 = _leaky(_mm(h1, we2_v[...]) + be2_ref[...])
    hp = h2[0:B]
    for c in range(1, C):
        hp = jnp.maximum(hp, h2[c * B:(c + 1) * B])
    cps[3].wait()
    h = _mm(hp, wef_v[...]) + bef_ref[...]
    cps[4].wait()
    i1 = _mm(h, wi1_v[...]) + bi1_ref[...]
    mu = i1[:, :Z // 4]
    mu_ref[...] = mu
    lv_ref[...] = i1[:, Z // 4:]
    cps[5].wait()
    cps[6].wait()
    p = _mm(_leaky(_mm(mu, wp1_v[...]) + bp1_ref[...]),
            wp2_v[...]) + bp2_ref[...]
    p_mu = p[:, :3 * Z // 4]
    plv_ref[...] = p[:, 3 * Z // 4:]
    cps[7].wait()
    cps[8].wait()
    qh = _leaky(_mm(jnp.concatenate([mu, h], axis=1), wq1_v[...])
                + bq1_ref[...])
    q = _mm(qh, wq2_v[...]) + bq2_ref[...]
    d_mu = q[:, :3 * Z // 4]
    dmu_ref[...] = d_mu
    dlv_ref[...] = q[:, 3 * Z // 4:]
    z2 = d_mu + p_mu
    cps[9].wait()
    d1 = _leaky(_mm(z2, wd1_v[...]) + bd1_ref[...])
    cps[10].wait()
    cw = _mm(d1, wd2_v[...]) + bd2_ref[...]
    cw_ref[...] = cw
    iota = jax.lax.broadcasted_iota(jnp.int32, (B, K), 1)
    for c in range(C):
        xc = cw[:, c * E:(c + 1) * E]
        x2 = jnp.sum(xc * xc, axis=1, keepdims=True)
        bookcps[c].wait()
        xb = _mm(xc, book_v[c])
        dist = x2 - 2.0 * xb + b2_ref[c:c + 1, :]
        dist_ref[:, c * K:(c + 1) * K] = dist
        mn = jnp.min(dist, axis=1, keepdims=True)
        idx_ref[:, c:c + 1] = jnp.min(
            jnp.where(dist == mn, iota, K), axis=1, keepdims=True)


def kernel(x, codebook, W_e1, b_e1, W_e2, b_e2, W_ef, b_ef, W_i1, b_i1,
           W_p1, b_p1, W_p2, b_p2, W_q1, b_q1, W_q2, b_q2, W_d1, b_d1,
           W_d2, b_d2):
    f32 = jnp.float32
    xr = x.reshape(B, C, E).transpose(1, 0, 2).reshape(C * B, E)
    b2 = jnp.sum(codebook ** 2, axis=-1)
    args = (
        xr, codebook, b2,
        W_e1, b_e1.reshape(1, -1), W_e2, b_e2.reshape(1, -1),
        W_ef, b_ef.reshape(1, -1),
        W_i1, b_i1.reshape(1, -1),
        W_p1, b_p1.reshape(1, -1), W_p2, b_p2.reshape(1, -1),
        W_q1, b_q1.reshape(1, -1),
        W_q2, b_q2.reshape(1, -1),
        W_d1, b_d1.reshape(1, -1), W_d2, b_d2.reshape(1, -1),
    )
    any_spec = pl.BlockSpec(memory_space=pltpu.MemorySpace.HBM)
    vmem_spec = pl.BlockSpec(memory_space=pltpu.MemorySpace.VMEM)
    in_specs = [any_spec, any_spec, vmem_spec]
    for _ in range(10):
        in_specs += [any_spec, vmem_spec]
    out_shape = [
        jax.ShapeDtypeStruct((B, CW), f32),
        jax.ShapeDtypeStruct((B, C * K), f32),
        jax.ShapeDtypeStruct((B, C), jnp.int32),
        jax.ShapeDtypeStruct((B, Z // 4), f32),
        jax.ShapeDtypeStruct((B, Z // 4), f32),
        jax.ShapeDtypeStruct((B, 3 * Z // 4), f32),
        jax.ShapeDtypeStruct((B, 3 * Z // 4), f32),
        jax.ShapeDtypeStruct((B, 3 * Z // 4), f32),
    ]
    scratch_shapes = [
        pltpu.VMEM((C * B, E), f32),
        pltpu.VMEM((Z, E), f32),
        pltpu.VMEM((Z, Z), f32),
        pltpu.VMEM((2 * Z, Z), f32),
        pltpu.VMEM((Z // 2, 2 * Z), f32),
        pltpu.VMEM((2 * Z, Z // 4), f32),
        pltpu.VMEM((3 * Z // 2, 2 * Z), f32),
        pltpu.VMEM((2 * Z, 9 * Z // 4), f32),
        pltpu.VMEM((3 * Z // 2, 2 * Z), f32),
        pltpu.VMEM((Z, 3 * Z // 4), f32),
        pltpu.VMEM((CW, Z), f32),
        pltpu.VMEM((C, K, E), f32),
        pltpu.SemaphoreType.DMA((11 + C,)),
    ]
    cw, dist, idx, mu, lv, plv, dmu, dlv = pl.pallas_call(
        _fwd,
        in_specs=in_specs,
        out_shape=out_shape,
        scratch_shapes=scratch_shapes,
        compiler_params=pltpu.CompilerParams(
            vmem_limit_bytes=100 * 1024 * 1024),
    )(*args)
    return (cw, dist.reshape(B, C, K), idx.reshape(-1, 1), mu, lv,
            plv, dmu, dlv)


# speedup vs baseline: 1.4233x; 1.4233x over previous
import jax
import jax.numpy as jnp
from jax.experimental import pallas as pl
from jax.experimental.pallas import tpu as pltpu

B = 64
C = 16
K = 1024
E = 256
CW = C * E
Z = 512


def _leaky(v):
    return jnp.where(v >= 0, v, 0.2 * v)


def _mm(a, b):
    return jax.lax.dot_general(a, b, (((1,), (1,)), ((), ())),
                               preferred_element_type=jnp.float32)


def _fwd(x_hbm, book_hbm,
         we1_hbm, be1_ref, we2_hbm, be2_ref, wef_hbm, bef_ref,
         wi1_hbm, bi1_ref, wp1_hbm, bp1_ref, wp2_hbm, bp2_ref,
         wq1_hbm, bq1_ref, wq2_hbm, bq2_ref,
         wd1_hbm, bd1_ref, wd2_hbm, bd2_ref,
         cw_ref, dist_ref, idx_ref, mu_ref, lv_ref, plv_ref,
         dmu_ref, dlv_ref,
         xr_v, we1_v, we2_v, wef_v, wi1_v, wp1_v, wp2_v, wq1_v, wq2_v,
         wd1_v, wd2_v, book_v, sems):
    xcps = [pltpu.make_async_copy(x_hbm.at[:, pl.ds(c * E, E)],
                                  xr_v.at[pl.ds(c * B, B), :],
                                  sems.at[c])
            for c in range(C)]
    pairs = [(we1_hbm, we1_v), (we2_hbm, we2_v),
             (wef_hbm, wef_v), (wi1_hbm, wi1_v), (wp1_hbm, wp1_v),
             (wp2_hbm, wp2_v), (wq1_hbm, wq1_v), (wq2_hbm, wq2_v),
             (wd1_hbm, wd1_v), (wd2_hbm, wd2_v)]
    cps = [pltpu.make_async_copy(s, d, sems.at[C + i])
           for i, (s, d) in enumerate(pairs)]
    bookcps = [pltpu.make_async_copy(book_hbm.at[c], book_v.at[c],
                                     sems.at[C + len(pairs) + c])
               for c in range(C)]
    for cp in xcps:
        cp.start()
    for cp in cps:
        cp.start()
    for cp in bookcps:
        cp.start()

    for cp in xcps:
        cp.wait()
    cps[0].wait()
    h1 = _leaky(_mm(xr_v[...], we1_v[...]) + be1_ref[...])
    cps[1].wait()
    h2 = _leaky(_mm(h1, we2_v[...]) + be2_ref[...])
    hp = h2[0:B]
    for c in range(1, C):
        hp = jnp.maximum(hp, h2[c * B:(c + 1) * B])
    cps[2].wait()
    h = _mm(hp, wef_v[...]) + bef_ref[...]
    cps[3].wait()
    i1 = _mm(h, wi1_v[...]) + bi1_ref[...]
    mu = i1[:, :Z // 4]
    mu_ref[...] = mu
    lv_ref[...] = i1[:, Z // 4:]
    cps[4].wait()
    cps[5].wait()
    p = _mm(_leaky(_mm(mu, wp1_v[...]) + bp1_ref[...]),
            wp2_v[...]) + bp2_ref[...]
    p_mu = p[:, :3 * Z // 4]
    plv_ref[...] = p[:, 3 * Z // 4:]
    cps[6].wait()
    cps[7].wait()
    qh = _leaky(_mm(jnp.concatenate([mu, h], axis=1), wq1_v[...])
                + bq1_ref[...])
    q = _mm(qh, wq2_v[...]) + bq2_ref[...]
    d_mu = q[:, :3 * Z // 4]
    dmu_ref[...] = d_mu
    dlv_ref[...] = q[:, 3 * Z // 4:]
    z2 = d_mu + p_mu
    cps[8].wait()
    d1 = _leaky(_mm(z2, wd1_v[...]) + bd1_ref[...])
    cps[9].wait()
    cw = _mm(d1, wd2_v[...]) + bd2_ref[...]
    cw_ref[...] = cw
    iota = jax.lax.broadcasted_iota(jnp.int32, (B, K), 1)
    ones = jnp.ones((1, E), jnp.float32)
    for c in range(C):
        xc = cw[:, c * E:(c + 1) * E]
        x2 = jnp.sum(xc * xc, axis=1, keepdims=True)
        bookcps[c].wait()
        bc = book_v[c]
        b2 = _mm(ones, bc * bc)
        xb = _mm(xc, bc)
        dist = x2 - 2.0 * xb + b2
        dist_ref[:, c, :] = dist
        mn = jnp.min(dist, axis=1, keepdims=True)
        idx_ref[:, c:c + 1] = jnp.min(
            jnp.where(dist == mn, iota, K), axis=1, keepdims=True)


def kernel(x, codebook, W_e1, b_e1, W_e2, b_e2, W_ef, b_ef, W_i1, b_i1,
           W_p1, b_p1, W_p2, b_p2, W_q1, b_q1, W_q2, b_q2, W_d1, b_d1,
           W_d2, b_d2):
    f32 = jnp.float32
    args = (
        x, codebook,
        W_e1, b_e1.reshape(1, -1), W_e2, b_e2.reshape(1, -1),
        W_ef, b_ef.reshape(1, -1),
        W_i1, b_i1.reshape(1, -1),
        W_p1, b_p1.reshape(1, -1), W_p2, b_p2.reshape(1, -1),
        W_q1, b_q1.reshape(1, -1),
        W_q2, b_q2.reshape(1, -1),
        W_d1, b_d1.reshape(1, -1), W_d2, b_d2.reshape(1, -1),
    )
    hbm_spec = pl.BlockSpec(memory_space=pltpu.MemorySpace.HBM)
    vmem_spec = pl.BlockSpec(memory_space=pltpu.MemorySpace.VMEM)
    in_specs = [hbm_spec, hbm_spec]
    for _ in range(10):
        in_specs += [hbm_spec, vmem_spec]
    out_shape = [
        jax.ShapeDtypeStruct((B, CW), f32),
        jax.ShapeDtypeStruct((B, C, K), f32),
        jax.ShapeDtypeStruct((B, C), jnp.int32),
        jax.ShapeDtypeStruct((B, Z // 4), f32),
        jax.ShapeDtypeStruct((B, Z // 4), f32),
        jax.ShapeDtypeStruct((B, 3 * Z // 4), f32),
        jax.ShapeDtypeStruct((B, 3 * Z // 4), f32),
        jax.ShapeDtypeStruct((B, 3 * Z // 4), f32),
    ]
    scratch_shapes = [
        pltpu.VMEM((C * B, E), f32),
        pltpu.VMEM((Z, E), f32),
        pltpu.VMEM((Z, Z), f32),
        pltpu.VMEM((2 * Z, Z), f32),
        pltpu.VMEM((Z // 2, 2 * Z), f32),
        pltpu.VMEM((2 * Z, Z // 4), f32),
        pltpu.VMEM((3 * Z // 2, 2 * Z), f32),
        pltpu.VMEM((2 * Z, 9 * Z // 4), f32),
        pltpu.VMEM((3 * Z // 2, 2 * Z), f32),
        pltpu.VMEM((Z, 3 * Z // 4), f32),
        pltpu.VMEM((CW, Z), f32),
        pltpu.VMEM((C, K, E), f32),
        pltpu.SemaphoreType.DMA((C + 10 + C,)),
    ]
    cw, dist, idx, mu, lv, plv, dmu, dlv = pl.pallas_call(
        _fwd,
        in_specs=in_specs,
        out_shape=out_shape,
        scratch_shapes=scratch_shapes,
        compiler_params=pltpu.CompilerParams(
            vmem_limit_bytes=100 * 1024 * 1024),
    )(*args)
    return (cw, dist, idx.reshape(-1, 1), mu, lv, plv, dmu, dlv)


# speedup vs baseline: 1.4991x; 1.0533x over previous
import jax
import jax.numpy as jnp
from jax.experimental import pallas as pl
from jax.experimental.pallas import tpu as pltpu

B = 64
C = 16
K = 1024
E = 256
CW = C * E
Z = 512


def _leaky(v):
    return jnp.where(v >= 0, v, 0.2 * v)


def _mm(a, b):
    return jax.lax.dot_general(a, b, (((1,), (1,)), ((), ())),
                               preferred_element_type=jnp.float32)


def _fwd(x_hbm, book_hbm,
         we1_hbm, be1_ref, we2_hbm, be2_ref, wef_hbm, bef_ref,
         wi1_hbm, bi1_ref, wp1_hbm, bp1_ref, wp2_hbm, bp2_ref,
         wq1_hbm, bq1_ref, wq2_hbm, bq2_ref,
         wd1_hbm, bd1_ref, wd2_hbm, bd2_ref,
         cw_ref, dist_ref, idx_ref, mu_ref, lv_ref, plv_ref,
         dmu_ref, dlv_ref,
         xr_v, we1_v, we2_v, wef_v, wi1_v, wp1_v, wp2_v, wq1_v, wq2_v,
         wd1_v, wd2_v, book_v, sems):
    xcps = [pltpu.make_async_copy(x_hbm.at[:, pl.ds(c * E, E)],
                                  xr_v.at[pl.ds(c * B, B), :],
                                  sems.at[c])
            for c in range(C)]
    pairs = [(we1_hbm, we1_v), (we2_hbm, we2_v),
             (wef_hbm, wef_v), (wi1_hbm, wi1_v), (wp1_hbm, wp1_v),
             (wp2_hbm, wp2_v), (wq1_hbm, wq1_v), (wq2_hbm, wq2_v),
             (wd1_hbm, wd1_v), (wd2_hbm, wd2_v)]
    cps = [pltpu.make_async_copy(s, d, sems.at[C + i])
           for i, (s, d) in enumerate(pairs)]
    bookcps = [pltpu.make_async_copy(book_hbm.at[c], book_v.at[c],
                                     sems.at[C + len(pairs) + c])
               for c in range(C)]
    for cp in xcps:
        cp.start()
    for cp in cps:
        cp.start()
    for cp in bookcps:
        cp.start()

    for cp in xcps:
        cp.wait()
    cps[0].wait()
    h1 = _leaky(_mm(xr_v[...], we1_v[...]) + be1_ref[...])
    cps[1].wait()
    h2 = _leaky(_mm(h1, we2_v[...]) + be2_ref[...])
    hp = h2[0:B]
    for c in range(1, C):
        hp = jnp.maximum(hp, h2[c * B:(c + 1) * B])
    cps[2].wait()
    h = _mm(hp, wef_v[...]) + bef_ref[...]
    cps[3].wait()
    i1 = _mm(h, wi1_v[...]) + bi1_ref[...]
    mu = i1[:, :Z // 4]
    mu_ref[...] = mu
    lv_ref[...] = i1[:, Z // 4:]
    cps[4].wait()
    cps[5].wait()
    p = _mm(_leaky(_mm(mu, wp1_v[...]) + bp1_ref[...]),
            wp2_v[...]) + bp2_ref[...]
    p_mu = p[:, :3 * Z // 4]
    plv_ref[...] = p[:, 3 * Z // 4:]
    cps[6].wait()
    cps[7].wait()
    qh = _leaky(_mm(jnp.concatenate([mu, h], axis=1), wq1_v[...])
                + bq1_ref[...])
    q = _mm(qh, wq2_v[...]) + bq2_ref[...]
    d_mu = q[:, :3 * Z // 4]
    dmu_ref[...] = d_mu
    dlv_ref[...] = q[:, 3 * Z // 4:]
    z2 = d_mu + p_mu
    cps[8].wait()
    d1 = _leaky(_mm(z2, wd1_v[...]) + bd1_ref[...])
    cps[9].wait()
    cw = _mm(d1, wd2_v[...]) + bd2_ref[...]
    cw_ref[...] = cw
    iota = jax.lax.broadcasted_iota(jnp.int32, (B, K), 1)
    for c in range(C):
        xc = cw[:, c * E:(c + 1) * E]
        x2 = jnp.sum(xc * xc, axis=1, keepdims=True)
        bookcps[c].wait()
        bc = book_v[c]
        bc3 = book_v[c:c + 1]
        b2 = jnp.sum(bc3 * bc3, axis=-1)
        xb = _mm(xc, bc)
        dist = x2 - 2.0 * xb + b2
        dist_ref[:, c, :] = dist
        mn = jnp.min(dist, axis=1, keepdims=True)
        idx_ref[:, c:c + 1] = jnp.min(
            jnp.where(dist == mn, iota, K), axis=1, keepdims=True)


def kernel(x, codebook, W_e1, b_e1, W_e2, b_e2, W_ef, b_ef, W_i1, b_i1,
           W_p1, b_p1, W_p2, b_p2, W_q1, b_q1, W_q2, b_q2, W_d1, b_d1,
           W_d2, b_d2):
    f32 = jnp.float32
    args = (
        x, codebook,
        W_e1, b_e1.reshape(1, -1), W_e2, b_e2.reshape(1, -1),
        W_ef, b_ef.reshape(1, -1),
        W_i1, b_i1.reshape(1, -1),
        W_p1, b_p1.reshape(1, -1), W_p2, b_p2.reshape(1, -1),
        W_q1, b_q1.reshape(1, -1),
        W_q2, b_q2.reshape(1, -1),
        W_d1, b_d1.reshape(1, -1), W_d2, b_d2.reshape(1, -1),
    )
    hbm_spec = pl.BlockSpec(memory_space=pltpu.MemorySpace.HBM)
    vmem_spec = pl.BlockSpec(memory_space=pltpu.MemorySpace.VMEM)
    in_specs = [hbm_spec, hbm_spec]
    for _ in range(10):
        in_specs += [hbm_spec, vmem_spec]
    out_shape = [
        jax.ShapeDtypeStruct((B, CW), f32),
        jax.ShapeDtypeStruct((B, C, K), f32),
        jax.ShapeDtypeStruct((B, C), jnp.int32),
        jax.ShapeDtypeStruct((B, Z // 4), f32),
        jax.ShapeDtypeStruct((B, Z // 4), f32),
        jax.ShapeDtypeStruct((B, 3 * Z // 4), f32),
        jax.ShapeDtypeStruct((B, 3 * Z // 4), f32),
        jax.ShapeDtypeStruct((B, 3 * Z // 4), f32),
    ]
    scratch_shapes = [
        pltpu.VMEM((C * B, E), f32),
        pltpu.VMEM((Z, E), f32),
        pltpu.VMEM((Z, Z), f32),
        pltpu.VMEM((2 * Z, Z), f32),
        pltpu.VMEM((Z // 2, 2 * Z), f32),
        pltpu.VMEM((2 * Z, Z // 4), f32),
        pltpu.VMEM((3 * Z // 2, 2 * Z), f32),
        pltpu.VMEM((2 * Z, 9 * Z // 4), f32),
        pltpu.VMEM((3 * Z // 2, 2 * Z), f32),
        pltpu.VMEM((Z, 3 * Z // 4), f32),
        pltpu.VMEM((CW, Z), f32),
        pltpu.VMEM((C, K, E), f32),
        pltpu.SemaphoreType.DMA((C + 10 + C,)),
    ]
    cw, dist, idx, mu, lv, plv, dmu, dlv = pl.pallas_call(
        _fwd,
        in_specs=in_specs,
        out_shape=out_shape,
        scratch_shapes=scratch_shapes,
        compiler_params=pltpu.CompilerParams(
            vmem_limit_bytes=100 * 1024 * 1024),
    )(*args)
    return (cw, dist, idx.reshape(-1, 1), mu, lv, plv, dmu, dlv)


# speedup vs baseline: 1.6805x; 1.1209x over previous
import jax
import jax.numpy as jnp
from jax.experimental import pallas as pl
from jax.experimental.pallas import tpu as pltpu

B = 64
C = 16
K = 1024
E = 256
CW = C * E
Z = 512


def _leaky(v):
    return jnp.where(v >= 0, v, 0.2 * v)


def _mm(a, b):
    return jax.lax.dot_general(a, b, (((1,), (1,)), ((), ())),
                               preferred_element_type=jnp.float32)


def _fwd(x_hbm, book_hbm,
         we1_hbm, be1_ref, we2_hbm, be2_ref, wef_hbm, bef_ref,
         wi1_hbm, bi1_ref, wp1_hbm, bp1_ref, wp2_hbm, bp2_ref,
         wq1_hbm, bq1_ref, wq2_hbm, bq2_ref,
         wd1_hbm, bd1_ref, wd2_hbm, bd2_ref,
         cw_hbm, dist_hbm, idx_ref, mu_ref, lv_ref, plv_ref,
         dmu_ref, dlv_ref,
         xr_v, we1_v, we2_v, wef_v, wi1_v, wp1_v, wp2_v, wq1_v, wq2_v,
         wd1_v, wd2_v, book_v, cw_v, dist_v, sems, osems):
    xcps = [pltpu.make_async_copy(x_hbm.at[:, pl.ds(c * E, E)],
                                  xr_v.at[pl.ds(c * B, B), :],
                                  sems.at[c])
            for c in range(C)]
    pairs = [(we1_hbm, we1_v), (we2_hbm, we2_v),
             (wef_hbm, wef_v), (wi1_hbm, wi1_v), (wp1_hbm, wp1_v),
             (wp2_hbm, wp2_v), (wq1_hbm, wq1_v), (wq2_hbm, wq2_v),
             (wd1_hbm, wd1_v), (wd2_hbm, wd2_v)]
    cps = [pltpu.make_async_copy(s, d, sems.at[C + i])
           for i, (s, d) in enumerate(pairs)]
    bookcps = [pltpu.make_async_copy(book_hbm.at[c], book_v.at[c],
                                     sems.at[C + len(pairs) + c])
               for c in range(C)]
    for cp in xcps:
        cp.start()
    for cp in cps:
        cp.start()
    for cp in bookcps:
        cp.start()

    for cp in xcps:
        cp.wait()
    cps[0].wait()
    h1 = _leaky(_mm(xr_v[...], we1_v[...]) + be1_ref[...])
    cps[1].wait()
    h2 = _leaky(_mm(h1, we2_v[...]) + be2_ref[...])
    hp = h2[0:B]
    for c in range(1, C):
        hp = jnp.maximum(hp, h2[c * B:(c + 1) * B])
    cps[2].wait()
    h = _mm(hp, wef_v[...]) + bef_ref[...]
    cps[3].wait()
    i1 = _mm(h, wi1_v[...]) + bi1_ref[...]
    mu = i1[:, :Z // 4]
    mu_ref[...] = mu
    lv_ref[...] = i1[:, Z // 4:]
    cps[4].wait()
    cps[5].wait()
    p = _mm(_leaky(_mm(mu, wp1_v[...]) + bp1_ref[...]),
            wp2_v[...]) + bp2_ref[...].reshape(1, 3 * Z // 2)
    p_mu = p[:, :3 * Z // 4]
    plv_ref[...] = p[:, 3 * Z // 4:]
    cps[6].wait()
    cps[7].wait()
    qh = _leaky(_mm(jnp.concatenate([mu, h], axis=1), wq1_v[...])
                + bq1_ref[...])
    q = _mm(qh, wq2_v[...]) + bq2_ref[...].reshape(1, 3 * Z // 2)
    d_mu = q[:, :3 * Z // 4]
    dmu_ref[...] = d_mu
    dlv_ref[...] = q[:, 3 * Z // 4:]
    z2 = d_mu + p_mu
    cps[8].wait()
    d1 = _leaky(_mm(z2, wd1_v[...]) + bd1_ref[...])
    cps[9].wait()
    cw = _mm(d1, wd2_v[...]) + bd2_ref[...]
    cw_v[...] = cw
    cwcp = pltpu.make_async_copy(cw_v, cw_hbm, osems.at[0])
    cwcp.start()
    iota = jax.lax.broadcasted_iota(jnp.int32, (B, K), 1)
    ocps = []
    for c in range(C):
        xc = cw[:, c * E:(c + 1) * E]
        x2 = jnp.sum(xc * xc, axis=1, keepdims=True)
        bookcps[c].wait()
        bc = book_v[c]
        bc3 = book_v[c:c + 1]
        b2 = jnp.sum(bc3 * bc3, axis=-1)
        xb = _mm(xc, bc)
        dist = x2 - 2.0 * xb + b2
        if c >= 2:
            ocps[c - 2].wait()
        dist_v[c % 2] = dist
        ocp = pltpu.make_async_copy(dist_v.at[c % 2],
                                    dist_hbm.at[:, c, :],
                                    osems.at[1 + c % 2])
        ocp.start()
        ocps.append(ocp)
        mn = jnp.min(dist, axis=1, keepdims=True)
        idx_ref[:, c:c + 1] = jnp.min(
            jnp.where(dist == mn, iota, K), axis=1, keepdims=True)
    cwcp.wait()
    ocps[C - 2].wait()
    ocps[C - 1].wait()


def kernel(x, codebook, W_e1, b_e1, W_e2, b_e2, W_ef, b_ef, W_i1, b_i1,
           W_p1, b_p1, W_p2, b_p2, W_q1, b_q1, W_q2, b_q2, W_d1, b_d1,
           W_d2, b_d2):
    f32 = jnp.float32
    args = (
        x, codebook,
        W_e1, b_e1.reshape(1, -1), W_e2, b_e2.reshape(1, -1),
        W_ef, b_ef.reshape(1, -1),
        W_i1, b_i1.reshape(1, -1),
        W_p1, b_p1.reshape(1, -1), W_p2, b_p2,
        W_q1, b_q1.reshape(1, -1),
        W_q2, b_q2,
        W_d1, b_d1.reshape(1, -1), W_d2, b_d2.reshape(1, -1),
    )
    hbm_spec = pl.BlockSpec(memory_space=pltpu.MemorySpace.HBM)
    vmem_spec = pl.BlockSpec(memory_space=pltpu.MemorySpace.VMEM)
    in_specs = [hbm_spec, hbm_spec]
    for _ in range(10):
        in_specs += [hbm_spec, vmem_spec]
    out_shape = [
        jax.ShapeDtypeStruct((B, CW), f32),
        jax.ShapeDtypeStruct((B, C, K), f32),
        jax.ShapeDtypeStruct((B, C), jnp.int32),
        jax.ShapeDtypeStruct((B, Z // 4), f32),
        jax.ShapeDtypeStruct((B, Z // 4), f32),
        jax.ShapeDtypeStruct((B, 3 * Z // 4), f32),
        jax.ShapeDtypeStruct((B, 3 * Z // 4), f32),
        jax.ShapeDtypeStruct((B, 3 * Z // 4), f32),
    ]
    out_specs = [hbm_spec, hbm_spec, vmem_spec, vmem_spec, vmem_spec,
                 vmem_spec, vmem_spec, vmem_spec]
    scratch_shapes = [
        pltpu.VMEM((C * B, E), f32),
        pltpu.VMEM((Z, E), f32),
        pltpu.VMEM((Z, Z), f32),
        pltpu.VMEM((2 * Z, Z), f32),
        pltpu.VMEM((Z // 2, 2 * Z), f32),
        pltpu.VMEM((2 * Z, Z // 4), f32),
        pltpu.VMEM((3 * Z // 2, 2 * Z), f32),
        pltpu.VMEM((2 * Z, 9 * Z // 4), f32),
        pltpu.VMEM((3 * Z // 2, 2 * Z), f32),
        pltpu.VMEM((Z, 3 * Z // 4), f32),
        pltpu.VMEM((CW, Z), f32),
        pltpu.VMEM((C, K, E), f32),
        pltpu.VMEM((B, CW), f32),
        pltpu.VMEM((2, B, K), f32),
        pltpu.SemaphoreType.DMA((C + 10 + C,)),
        pltpu.SemaphoreType.DMA((3,)),
    ]
    cw, dist, idx, mu, lv, plv, dmu, dlv = pl.pallas_call(
        _fwd,
        in_specs=in_specs,
        out_shape=out_shape,
        out_specs=out_specs,
        scratch_shapes=scratch_shapes,
        compiler_params=pltpu.CompilerParams(
            vmem_limit_bytes=100 * 1024 * 1024),
    )(*args)
    return (cw, dist, idx.reshape(-1, 1), mu, lv, plv, dmu, dlv)
